# baseline (speedup 1.0000x reference)
import functools

import jax
import jax.numpy as jnp
from jax import lax
from jax.experimental import pallas as pl
from jax.experimental.pallas import tpu as pltpu
from jax.experimental.pallas import tpu_sc as plsc

L = 16
K = 128
NB = 4


@functools.lru_cache(maxsize=None)
def _make_program(n_rows, seq_len, n_seg, depth):
    info = plsc.get_sparse_core_info()
    nw = info.num_cores * info.num_subcores
    assert n_rows % (nw * NB * K) == 0
    rows_per_w = n_rows // nw
    n_chunks = rows_per_w // K
    assert n_chunks >= 2 * NB
    cr = n_seg * seq_len

    mesh = plsc.VectorSubcoreMesh(core_axis_name="c", subcore_axis_name="s")

    scratch = (
        [pltpu.VMEM((K,), jnp.int32) for _ in range(NB)] +
        [pltpu.VMEM((K,), jnp.int32) for _ in range(NB)] +
        [pltpu.VMEM((K, depth), jnp.float32) for _ in range(NB)] +
        [pltpu.VMEM((cr, depth), jnp.float32),
         pltpu.VMEM((n_seg, depth), jnp.float32)] +
        [pltpu.SemaphoreType.DMA for _ in range(4 * NB)]
    )

    @functools.partial(
        pl.kernel,
        out_type=jax.ShapeDtypeStruct((n_rows, depth), jnp.float32),
        mesh=mesh,
        compiler_params=pltpu.CompilerParams(needs_layout_passes=False),
        scratch_types=scratch,
    )
    def prog(x_hbm, g_hbm, tok_hbm, pos_hbm, seg_hbm, out_hbm, *bufs):
        idxs = bufs[0:NB]
        gids = bufs[NB:2 * NB]
        rows = bufs[2 * NB:3 * NB]
        comb_v, seg_v = bufs[3 * NB], bufs[3 * NB + 1]
        sems = bufs[3 * NB + 2:]
        sin_x, sin_g = sems[0:NB], sems[NB:2 * NB]
        sg, so = sems[2 * NB:3 * NB], sems[3 * NB:4 * NB]

        wid = lax.axis_index("s") * info.num_cores + lax.axis_index("c")
        iota = lax.iota(jnp.int32, L)
        base0 = wid * rows_per_w

        for g in range(n_seg):
            pltpu.sync_copy(pos_hbm.at[pl.ds(0, seq_len)],
                            comb_v.at[pl.ds(g * seq_len, seq_len)])
        pltpu.sync_copy(seg_hbm, seg_v)

        @plsc.parallel_loop(0, cr, step=1, unroll=2)
        def seg_add_row(t):
            half = jnp.full((L,), t // seq_len, dtype=jnp.int32)
            rsplat = jnp.full((L,), t, dtype=jnp.int32)
            for d8 in range(depth // L):
                col = d8 * L + iota
                v = plsc.load_gather(seg_v, [half, col])
                plsc.addupdate_scatter(comb_v, [rsplat, col], v)

        def issue_in(c, p):
            base = base0 + c * K
            pltpu.async_copy(x_hbm.at[pl.ds(base, K)], idxs[p], sin_x[p])
            pltpu.async_copy(g_hbm.at[pl.ds(base, K)], gids[p], sin_g[p])

        def wait_in(p):
            pltpu.make_async_copy(x_hbm.at[pl.ds(0, K)], idxs[p], sin_x[p]).wait()
            pltpu.make_async_copy(g_hbm.at[pl.ds(0, K)], gids[p], sin_g[p]).wait()

        def issue_gather(p):
            pltpu.async_copy(tok_hbm.at[idxs[p]], rows[p], sg[p])

        def wait_gather(p):
            pltpu.make_async_copy(tok_hbm.at[idxs[p]], rows[p], sg[p]).wait()

        def issue_out(c, p):
            base = base0 + c * K
            pltpu.async_copy(rows[p], out_hbm.at[pl.ds(base, K)], so[p])

        def wait_out(p):
            pltpu.make_async_copy(rows[p], out_hbm.at[pl.ds(0, K)], so[p]).wait()

        def compute(c, p):
            base = base0 + c * K
            for j in range(K // L):
                gv = gids[p][pl.ds(j * L, L)]
                svec = (base + j * L + iota) % seq_len
                comb = gv * seq_len + svec

                @plsc.parallel_loop(0, L, step=1, unroll=4)
                def rbody(r):
                    ridx = jnp.full((L,), r, dtype=jnp.int32)
                    csplat = comb.at[ridx].get(mode="promise_in_bounds")
                    rsplat = jnp.full((L,), j * L + r, dtype=jnp.int32)
                    for d8 in range(depth // L):
                        col = d8 * L + iota
                        v = plsc.load_gather(comb_v, [csplat, col])
                        plsc.addupdate_scatter(rows[p], [rsplat, col], v)

        def body(c, p):
            q = (p + 1) % NB

            @pl.when(c + 1 < n_chunks)
            def _prep():
                wait_in(q)

                @pl.when(c >= NB - 1)
                def _wo():
                    wait_out(q)

                issue_gather(q)

            wait_gather(p)
            compute(c, p)

            @pl.when(c + NB < n_chunks)
            def _in():
                issue_in(c + NB, p)

            issue_out(c, p)

        for p in range(NB):
            issue_in(p, p)
        wait_in(0)
        issue_gather(0)

        def quad(t, _):
            for r in range(NB):
                body(NB * t + r, r)
            return 0

        lax.fori_loop(0, n_chunks // NB, quad, 0)
        for p in range(NB):
            wait_out(p)

    return prog


def kernel(x, segment_ids, token_table, pos_table, seg_table):
    b, s = x.shape
    _, depth = token_table.shape
    n_rows = b * s
    prog = _make_program(n_rows, s, seg_table.shape[0], depth)
    out = prog(x.reshape(n_rows).astype(jnp.int32),
               segment_ids.reshape(n_rows).astype(jnp.int32),
               token_table, pos_table, seg_table)
    return out.reshape(b, s, depth)


# speedup vs baseline: 1.2745x; 1.2745x over previous
import functools

import jax
import jax.numpy as jnp
from jax import lax
from jax.experimental import pallas as pl
from jax.experimental.pallas import tpu as pltpu
from jax.experimental.pallas import tpu_sc as plsc

L = 16
K = 128
NB = 4


@functools.lru_cache(maxsize=None)
def _make_program(n_rows, seq_len, n_seg, depth):
    info = plsc.get_sparse_core_info()
    nw = info.num_cores * info.num_subcores
    assert n_rows % (nw * NB * K) == 0
    rows_per_w = n_rows // nw
    n_chunks = rows_per_w // K
    assert n_chunks >= 2 * NB
    cr = n_seg * seq_len

    mesh = plsc.VectorSubcoreMesh(core_axis_name="c", subcore_axis_name="s")

    scratch = (
        [pltpu.VMEM((K,), jnp.int32) for _ in range(NB)] +
        [pltpu.VMEM((K,), jnp.int32) for _ in range(NB)] +
        [pltpu.VMEM((K, depth), jnp.float32) for _ in range(NB)] +
        [pltpu.VMEM((cr, depth), jnp.float32),
         pltpu.VMEM((n_seg, depth), jnp.float32)] +
        [pltpu.SemaphoreType.DMA for _ in range(4 * NB)]
    )

    @functools.partial(
        pl.kernel,
        out_type=jax.ShapeDtypeStruct((n_rows, depth), jnp.float32),
        mesh=mesh,
        compiler_params=pltpu.CompilerParams(needs_layout_passes=False),
        scratch_types=scratch,
    )
    def prog(x_hbm, g_hbm, tok_hbm, pos_hbm, seg_hbm, out_hbm, *bufs):
        idxs = bufs[0:NB]
        gids = bufs[NB:2 * NB]
        rows = bufs[2 * NB:3 * NB]
        comb_v, seg_v = bufs[3 * NB], bufs[3 * NB + 1]
        sems = bufs[3 * NB + 2:]
        sin_x, sin_g = sems[0:NB], sems[NB:2 * NB]
        sg, so = sems[2 * NB:3 * NB], sems[3 * NB:4 * NB]

        wid = lax.axis_index("s") * info.num_cores + lax.axis_index("c")
        iota = lax.iota(jnp.int32, L)
        base0 = wid * rows_per_w

        for g in range(n_seg):
            pltpu.sync_copy(pos_hbm.at[pl.ds(0, seq_len)],
                            comb_v.at[pl.ds(g * seq_len, seq_len)])
        pltpu.sync_copy(seg_hbm, seg_v)

        for g in range(n_seg):
            segrow = [seg_v[g, pl.ds(d8 * L, L)] for d8 in range(depth // L)]

            @plsc.parallel_loop(g * seq_len, (g + 1) * seq_len, step=1, unroll=2)
            def seg_add_row(t):
                rsplat = jnp.full((L,), t, dtype=jnp.int32)
                for d8 in range(depth // L):
                    plsc.addupdate_scatter(comb_v, [rsplat, d8 * L + iota],
                                           segrow[d8])

        def issue_in(c, p):
            base = base0 + c * K
            pltpu.async_copy(x_hbm.at[pl.ds(base, K)], idxs[p], sin_x[p])
            pltpu.async_copy(g_hbm.at[pl.ds(base, K)], gids[p], sin_g[p])

        def wait_in(p):
            pltpu.make_async_copy(x_hbm.at[pl.ds(0, K)], idxs[p], sin_x[p]).wait()
            pltpu.make_async_copy(g_hbm.at[pl.ds(0, K)], gids[p], sin_g[p]).wait()

        def issue_gather(p):
            pltpu.async_copy(tok_hbm.at[idxs[p]], rows[p], sg[p])

        def wait_gather(p):
            pltpu.make_async_copy(tok_hbm.at[idxs[p]], rows[p], sg[p]).wait()

        def issue_out(c, p):
            base = base0 + c * K
            pltpu.async_copy(rows[p], out_hbm.at[pl.ds(base, K)], so[p])

        def wait_out(p):
            pltpu.make_async_copy(rows[p], out_hbm.at[pl.ds(0, K)], so[p]).wait()

        def compute(c, p):
            base = base0 + c * K
            for j in range(K // L):
                gv = gids[p][pl.ds(j * L, L)]
                svec = (base + j * L + iota) % seq_len
                comb = gv * seq_len + svec

                @plsc.parallel_loop(0, L, step=1, unroll=2)
                def rbody(r):
                    ridx = jnp.full((L,), r, dtype=jnp.int32)
                    csplat = comb.at[ridx].get(mode="promise_in_bounds")
                    rsplat = jnp.full((L,), j * L + r, dtype=jnp.int32)
                    for d8 in range(depth // L):
                        col = d8 * L + iota
                        v = plsc.load_gather(comb_v, [csplat, col])
                        plsc.addupdate_scatter(rows[p], [rsplat, col], v)

        def body(c, p):
            q = (p + 1) % NB

            @pl.when(c + 1 < n_chunks)
            def _prep():
                wait_in(q)

                @pl.when(c >= NB - 1)
                def _wo():
                    wait_out(q)

                issue_gather(q)

            wait_gather(p)
            compute(c, p)

            @pl.when(c + NB < n_chunks)
            def _in():
                issue_in(c + NB, p)

            issue_out(c, p)

        for p in range(NB):
            issue_in(p, p)
        wait_in(0)
        issue_gather(0)

        def quad(t, _):
            for r in range(NB):
                body(NB * t + r, r)
            return 0

        lax.fori_loop(0, n_chunks // NB, quad, 0)
        for p in range(NB):
            wait_out(p)

    return prog


def kernel(x, segment_ids, token_table, pos_table, seg_table):
    b, s = x.shape
    _, depth = token_table.shape
    n_rows = b * s
    prog = _make_program(n_rows, s, seg_table.shape[0], depth)
    out = prog(x.reshape(n_rows).astype(jnp.int32),
               segment_ids.reshape(n_rows).astype(jnp.int32),
               token_table, pos_table, seg_table)
    return out.reshape(b, s, depth)


# speedup vs baseline: 1.2785x; 1.0031x over previous
import functools

import jax
import jax.numpy as jnp
from jax import lax
from jax.experimental import pallas as pl
from jax.experimental.pallas import tpu as pltpu
from jax.experimental.pallas import tpu_sc as plsc

L = 16
K = 128
NB = 4


@functools.lru_cache(maxsize=None)
def _make_program(n_rows, seq_len, n_seg, depth):
    info = plsc.get_sparse_core_info()
    nw = info.num_cores * info.num_subcores
    assert n_rows % (nw * NB * K) == 0
    rows_per_w = n_rows // nw
    n_chunks = rows_per_w // K
    assert n_chunks >= 2 * NB
    cr = n_seg * seq_len

    mesh = plsc.VectorSubcoreMesh(core_axis_name="c", subcore_axis_name="s")

    scratch = (
        [pltpu.VMEM((2 * K,), jnp.int32) for _ in range(NB)] +
        [pltpu.VMEM((K, depth), jnp.float32) for _ in range(NB)] +
        [pltpu.VMEM((cr, depth), jnp.float32),
         pltpu.VMEM((n_seg, depth), jnp.float32)] +
        [pltpu.SemaphoreType.DMA for _ in range(3 * NB)]
    )

    @functools.partial(
        pl.kernel,
        out_type=jax.ShapeDtypeStruct((n_rows, depth), jnp.float32),
        mesh=mesh,
        compiler_params=pltpu.CompilerParams(needs_layout_passes=False),
        scratch_types=scratch,
    )
    def prog(pk_hbm, tok_hbm, pos_hbm, seg_hbm, out_hbm, *bufs):
        pks = bufs[0:NB]
        rows = bufs[NB:2 * NB]
        comb_v, seg_v = bufs[2 * NB], bufs[2 * NB + 1]
        sems = bufs[2 * NB + 2:]
        sin = sems[0:NB]
        sg, so = sems[NB:2 * NB], sems[2 * NB:3 * NB]

        wid = lax.axis_index("s") * info.num_cores + lax.axis_index("c")
        iota = lax.iota(jnp.int32, L)
        base0 = wid * rows_per_w

        for g in range(n_seg):
            pltpu.sync_copy(pos_hbm.at[pl.ds(0, seq_len)],
                            comb_v.at[pl.ds(g * seq_len, seq_len)])
        pltpu.sync_copy(seg_hbm, seg_v)

        for g in range(n_seg):
            segrow = [seg_v[g, pl.ds(d8 * L, L)] for d8 in range(depth // L)]

            @plsc.parallel_loop(g * seq_len, (g + 1) * seq_len, step=1, unroll=2)
            def seg_add_row(t):
                rsplat = jnp.full((L,), t, dtype=jnp.int32)
                for d8 in range(depth // L):
                    plsc.addupdate_scatter(comb_v, [rsplat, d8 * L + iota],
                                           segrow[d8])

        chunk0 = base0 // K

        def issue_in(c, p):
            pltpu.async_copy(pk_hbm.at[chunk0 + c], pks[p], sin[p])

        def wait_in(p):
            pltpu.make_async_copy(pk_hbm.at[0], pks[p], sin[p]).wait()

        def issue_gather(p):
            pltpu.async_copy(tok_hbm.at[pks[p].at[pl.ds(0, K)]], rows[p], sg[p])

        def wait_gather(p):
            pltpu.make_async_copy(tok_hbm.at[pks[p].at[pl.ds(0, K)]], rows[p],
                                  sg[p]).wait()

        def issue_out(c, p):
            base = base0 + c * K
            pltpu.async_copy(rows[p], out_hbm.at[pl.ds(base, K)], so[p])

        def wait_out(p):
            pltpu.make_async_copy(rows[p], out_hbm.at[pl.ds(0, K)], so[p]).wait()

        def compute(c, p):
            base = base0 + c * K
            for j in range(K // L):
                gv = pks[p][pl.ds(K + j * L, L)]
                svec = (base + j * L + iota) % seq_len
                comb = gv * seq_len + svec

                @plsc.parallel_loop(0, L, step=1, unroll=2)
                def rbody(r):
                    ridx = jnp.full((L,), r, dtype=jnp.int32)
                    csplat = comb.at[ridx].get(mode="promise_in_bounds")
                    rsplat = jnp.full((L,), j * L + r, dtype=jnp.int32)
                    for d8 in range(depth // L):
                        col = d8 * L + iota
                        v = plsc.load_gather(comb_v, [csplat, col])
                        plsc.addupdate_scatter(rows[p], [rsplat, col], v)

        def body(c, p):
            q = (p + 1) % NB

            @pl.when(c + 1 < n_chunks)
            def _prep():
                wait_in(q)

                @pl.when(c >= NB - 1)
                def _wo():
                    wait_out(q)

                issue_gather(q)

            wait_gather(p)
            compute(c, p)

            @pl.when(c + NB < n_chunks)
            def _in():
                issue_in(c + NB, p)

            issue_out(c, p)

        for p in range(NB):
            issue_in(p, p)
        wait_in(0)
        issue_gather(0)

        def quad(t, _):
            for r in range(NB):
                body(NB * t + r, r)
            return 0

        lax.fori_loop(0, n_chunks // NB, quad, 0)
        for p in range(NB):
            wait_out(p)

    return prog


def kernel(x, segment_ids, token_table, pos_table, seg_table):
    b, s = x.shape
    _, depth = token_table.shape
    n_rows = b * s
    prog = _make_program(n_rows, s, seg_table.shape[0], depth)
    packed = jnp.concatenate(
        [x.reshape(n_rows // K, K).astype(jnp.int32),
         segment_ids.reshape(n_rows // K, K).astype(jnp.int32)], axis=1)
    out = prog(packed, token_table, pos_table, seg_table)
    return out.reshape(b, s, depth)


# speedup vs baseline: 1.3215x; 1.0336x over previous
import functools

import jax
import jax.numpy as jnp
from jax import lax
from jax.experimental import pallas as pl
from jax.experimental.pallas import tpu as pltpu
from jax.experimental.pallas import tpu_sc as plsc

L = 16
K = 128
NB = 4


@functools.lru_cache(maxsize=None)
def _make_program(n_rows, seq_len, n_seg, depth):
    info = plsc.get_sparse_core_info()
    nw = info.num_cores * info.num_subcores
    assert n_rows % (nw * NB * K) == 0
    rows_per_w = n_rows // nw
    n_chunks = rows_per_w // K
    assert n_chunks >= 2 * NB
    cr = n_seg * seq_len

    mesh = plsc.VectorSubcoreMesh(core_axis_name="c", subcore_axis_name="s")

    scratch = (
        [pltpu.VMEM((2 * K,), jnp.int32) for _ in range(NB)] +
        [pltpu.VMEM((K, depth), jnp.float32) for _ in range(NB)] +
        [pltpu.VMEM((cr, depth), jnp.float32),
         pltpu.VMEM((n_seg, depth), jnp.float32)] +
        [pltpu.SemaphoreType.DMA for _ in range(3 * NB)]
    )

    @functools.partial(
        pl.kernel,
        out_type=jax.ShapeDtypeStruct((n_rows, depth), jnp.float32),
        mesh=mesh,
        compiler_params=pltpu.CompilerParams(needs_layout_passes=False),
        scratch_types=scratch,
    )
    def prog(pk_hbm, tok_hbm, pos_hbm, seg_hbm, out_hbm, *bufs):
        pks = bufs[0:NB]
        rows = bufs[NB:2 * NB]
        comb_v, seg_v = bufs[2 * NB], bufs[2 * NB + 1]
        sems = bufs[2 * NB + 2:]
        sin = sems[0:NB]
        sg, so = sems[NB:2 * NB], sems[2 * NB:3 * NB]

        wid = lax.axis_index("s") * info.num_cores + lax.axis_index("c")
        iota = lax.iota(jnp.int32, L)
        base0 = wid * rows_per_w

        for g in range(n_seg):
            pltpu.sync_copy(pos_hbm.at[pl.ds(0, seq_len)],
                            comb_v.at[pl.ds(g * seq_len, seq_len)])
        pltpu.sync_copy(seg_hbm, seg_v)

        for g in range(n_seg):
            segrow = [seg_v[g, pl.ds(d8 * L, L)] for d8 in range(depth // L)]

            @plsc.parallel_loop(g * seq_len, (g + 1) * seq_len, step=1, unroll=2)
            def seg_add_row(t):
                rsplat = jnp.full((L,), t, dtype=jnp.int32)
                for d8 in range(depth // L):
                    plsc.addupdate_scatter(comb_v, [rsplat, d8 * L + iota],
                                           segrow[d8])

        chunk0 = base0 // K

        def issue_in(c, p):
            pltpu.async_copy(pk_hbm.at[chunk0 + c], pks[p], sin[p])

        def wait_in(p):
            pltpu.make_async_copy(pk_hbm.at[0], pks[p], sin[p]).wait()

        def issue_gather(p):
            pltpu.async_copy(tok_hbm.at[pks[p].at[pl.ds(0, K)]], rows[p], sg[p])

        def wait_gather(p):
            pltpu.make_async_copy(tok_hbm.at[pks[p].at[pl.ds(0, K)]], rows[p],
                                  sg[p]).wait()

        def issue_out(c, p):
            base = base0 + c * K
            pltpu.async_copy(rows[p], out_hbm.at[pl.ds(base, K)], so[p])

        def wait_out(p):
            pltpu.make_async_copy(rows[p], out_hbm.at[pl.ds(0, K)], so[p]).wait()

        def compute(c, p):
            base = base0 + c * K
            for j in range(K // L):
                gv = pks[p][pl.ds(K + j * L, L)]
                svec = (base + j * L + iota) % seq_len
                comb = gv * seq_len + svec

                @plsc.parallel_loop(0, L, step=1, unroll=2)
                def rbody(r):
                    ridx = jnp.full((L,), r, dtype=jnp.int32)
                    csplat = comb.at[ridx].get(mode="promise_in_bounds")
                    rsplat = jnp.full((L,), j * L + r, dtype=jnp.int32)
                    for d8 in range(depth // L):
                        col = d8 * L + iota
                        v = plsc.load_gather(comb_v, [csplat, col])
                        plsc.addupdate_scatter(rows[p], [rsplat, col], v)

        def body(c, p):
            q = (p + 2) % NB

            @pl.when(c + 2 < n_chunks)
            def _prep():
                wait_in(q)

                @pl.when(c >= 2)
                def _wo():
                    wait_out(q)

                issue_gather(q)

            wait_gather(p)
            compute(c, p)

            @pl.when(c + NB < n_chunks)
            def _in():
                issue_in(c + NB, p)

            issue_out(c, p)

        for p in range(NB):
            issue_in(p, p)
        wait_in(0)
        issue_gather(0)
        wait_in(1)
        issue_gather(1)

        def quad(t, _):
            for r in range(NB):
                body(NB * t + r, r)
            return 0

        lax.fori_loop(0, n_chunks // NB, quad, 0)
        for p in range(NB):
            wait_out(p)

    return prog


def kernel(x, segment_ids, token_table, pos_table, seg_table):
    b, s = x.shape
    _, depth = token_table.shape
    n_rows = b * s
    prog = _make_program(n_rows, s, seg_table.shape[0], depth)
    packed = jnp.concatenate(
        [x.reshape(n_rows // K, K).astype(jnp.int32),
         segment_ids.reshape(n_rows // K, K).astype(jnp.int32)], axis=1)
    out = prog(packed, token_table, pos_table, seg_table)
    return out.reshape(b, s, depth)
